# baseline (speedup 1.0000x reference)
import functools

import jax
import jax.numpy as jnp
from jax import lax
from jax.experimental import pallas as pl
from jax.experimental.pallas import tpu as pltpu
from jax.experimental.pallas import tpu_sc as plsc

_N = 10000
_E = 320000
_D = 128
_G = 64
_NC = 2
_NS = 16
_NW = _NC * _NS
_FW = _D // _NS
_EH = _E // _NC
_C = 128
_CH = 50
_NSC = _EH // (_CH * _C)
_EW = _E // _NW
_NPAD = 10240
_TBL = _NPAD * _FW
_NB = 1000
_GRID = _N // _NB

_sc_mesh = plsc.VectorSubcoreMesh(core_axis_name="c", subcore_axis_name="s")
_sc_params = pltpu.CompilerParams(needs_layout_passes=False,
                                  use_tc_tiling_on_sc=False)


def _deg_body(dst_hbm, out_hbm, idx_v, table):
    c = lax.axis_index("c")
    s = lax.axis_index("s")
    wid = c * _NS + s

    def _zero(i, carry):
        table[pl.ds(i * 16, 16)] = jnp.zeros((16,), jnp.float32)
        return carry

    lax.fori_loop(0, _NPAD // 16, _zero, 0)
    pltpu.sync_copy(dst_hbm.at[wid], idx_v)
    ones = jnp.ones((16,), jnp.float32)

    def _count(k, carry):
        dst16 = idx_v[pl.ds(k * 16, 16)]
        plsc.addupdate_scatter(table, [dst16], ones)
        return carry

    lax.fori_loop(0, _EW // 16, _count, 0)
    pltpu.sync_copy(table, out_hbm.at[wid])


_deg_call = pl.kernel(
    _deg_body,
    out_type=jax.ShapeDtypeStruct((_NW, _NPAD), jnp.float32),
    mesh=_sc_mesh,
    compiler_params=_sc_params,
    scratch_types=[
        pltpu.VMEM((_EW,), jnp.int32),
        pltpu.VMEM((_NPAD,), jnp.float32),
    ],
)


def _agg_body(hp_hbm, comb_hbm, out_hbm, cbuf, rows0, rows1, table, sem0, sem1):
    c = lax.axis_index("c")
    s = lax.axis_index("s")
    wid = c * _NS + s

    def _zero(i, carry):
        table[pl.ds(i * 16, 16)] = jnp.zeros((16,), jnp.float32)
        return carry

    lax.fori_loop(0, _TBL // 16, _zero, 0)

    iota16 = lax.iota(jnp.int32, 16)

    def _compute(j, rows):
        for g in range(_C // 16):
            base8 = cbuf[j, 1, pl.ds(g * 16, 16)]
            row16 = g * 16 + iota16
            for f in range(_FW):
                f16 = jnp.full((16,), f, jnp.int32)
                vals = plsc.load_gather(rows, [row16, f16])
                plsc.addupdate_scatter(table, [base8 + f], vals)

    def _gather(j, rows, sem):
        return pltpu.async_copy(hp_hbm.at[cbuf.at[j, 0]], rows, sem)

    def _super(sc, carry):
        pltpu.sync_copy(comb_hbm.at[wid, sc], cbuf)
        _gather(0, rows0, sem0)

        def _pair(jj, carry2):
            a = 2 * jj
            _gather(a + 1, rows1, sem1)
            pltpu.make_async_copy(hp_hbm.at[cbuf.at[a, 0]], rows0, sem0).wait()
            _compute(a, rows0)

            @pl.when(jj < _CH // 2 - 1)
            def _():
                _gather(a + 2, rows0, sem0)

            pltpu.make_async_copy(
                hp_hbm.at[cbuf.at[a + 1, 0]], rows1, sem1).wait()
            _compute(a + 1, rows1)
            return carry2

        lax.fori_loop(0, _CH // 2, _pair, 0)
        return carry

    lax.fori_loop(0, _NSC, _super, 0)
    pltpu.sync_copy(table, out_hbm.at[c, s])


_agg_call = pl.kernel(
    _agg_body,
    out_type=jax.ShapeDtypeStruct((_NC, _NS, _TBL), jnp.float32),
    mesh=_sc_mesh,
    compiler_params=_sc_params,
    scratch_types=[
        pltpu.VMEM((_CH, 2, _C), jnp.int32),
        pltpu.VMEM((_C, _FW), jnp.float32),
        pltpu.VMEM((_C, _FW), jnp.float32),
        pltpu.VMEM((_TBL,), jnp.float32),
        pltpu.SemaphoreType.DMA,
        pltpu.SemaphoreType.DMA,
    ],
)


def _lin_body(x_ref, wt_ref, b_ref, disb_ref, hp_ref):
    h = jnp.dot(x_ref[...], wt_ref[...], preferred_element_type=jnp.float32)
    hp_ref[...] = (h + b_ref[...]) * disb_ref[...]


_lin_call = pl.pallas_call(
    _lin_body,
    grid=(_GRID,),
    in_specs=[
        pl.BlockSpec((_NB, _D), lambda i: (i, 0)),
        pl.BlockSpec((_D, _D), lambda i: (0, 0)),
        pl.BlockSpec((1, _D), lambda i: (0, 0)),
        pl.BlockSpec((_NB, _D), lambda i: (i, 0)),
    ],
    out_specs=pl.BlockSpec((_NB, _D), lambda i: (i, 0)),
    out_shape=jax.ShapeDtypeStruct((_N, _D), jnp.float32),
)


def _conv_body(p0_ref, p1_ref, hp_ref, disb_ref, bias_ref, r_ref, st_ref, acc):
    i = pl.program_id(0)
    conv = disb_ref[...] * (p0_ref[0] + p1_ref[0] + hp_ref[...]) + bias_ref[...]
    r = jnp.maximum(conv, 0.0)
    r_ref[...] = r

    @pl.when(i == 0)
    def _():
        acc[...] = jnp.zeros_like(acc)

    acc[0:1, :] += jnp.sum(r, axis=0, keepdims=True)
    acc[1:2, :] += jnp.sum(r * r, axis=0, keepdims=True)

    @pl.when(i == _GRID - 1)
    def _():
        st_ref[...] = acc[...]


_conv_call = pl.pallas_call(
    _conv_body,
    grid=(_GRID,),
    in_specs=[
        pl.BlockSpec((1, _NB, _D), lambda i: (0, i, 0)),
        pl.BlockSpec((1, _NB, _D), lambda i: (1, i, 0)),
        pl.BlockSpec((_NB, _D), lambda i: (i, 0)),
        pl.BlockSpec((_NB, _D), lambda i: (i, 0)),
        pl.BlockSpec((1, _D), lambda i: (0, 0)),
    ],
    out_specs=[
        pl.BlockSpec((_NB, _D), lambda i: (i, 0)),
        pl.BlockSpec((8, _D), lambda i: (0, 0)),
    ],
    out_shape=[
        jax.ShapeDtypeStruct((_N, _D), jnp.float32),
        jax.ShapeDtypeStruct((8, _D), jnp.float32),
    ],
    scratch_shapes=[pltpu.VMEM((8, _D), jnp.float32)],
)


def _bn_math(r, st_ref, gamma_ref, beta_ref):
    m = st_ref[0:1, :] * (1.0 / _N)
    var = st_ref[1:2, :] * (1.0 / _N) - m * m
    a = lax.rsqrt(var + 1e-5) * gamma_ref[...]
    return r * a + (beta_ref[...] - m * a)


def _pool_block(z, batch_ref):
    bt = batch_ref[0]
    ids = lax.broadcasted_iota(jnp.int32, (_G, 1), 0)
    onehot = (bt == ids).astype(jnp.float32)
    return jnp.dot(onehot, z, preferred_element_type=jnp.float32)


def _bn_body(r_ref, st_ref, gamma_ref, beta_ref, batch_ref, wt_ref, bn_ref,
             disb_ref, z_ref, g_ref, h_ref, gacc):
    i = pl.program_id(0)
    z = _bn_math(r_ref[...], st_ref, gamma_ref, beta_ref)
    z_ref[...] = z

    @pl.when(i == 0)
    def _():
        gacc[...] = jnp.zeros_like(gacc)

    gacc[...] += _pool_block(z, batch_ref)

    @pl.when(i == _GRID - 1)
    def _():
        g_ref[...] = gacc[...]

    h = jnp.dot(z, wt_ref[...], preferred_element_type=jnp.float32)
    h_ref[...] = (h + bn_ref[...]) * disb_ref[...]


_bn_call = pl.pallas_call(
    _bn_body,
    grid=(_GRID,),
    in_specs=[
        pl.BlockSpec((_NB, _D), lambda i: (i, 0)),
        pl.BlockSpec((8, _D), lambda i: (0, 0)),
        pl.BlockSpec((1, _D), lambda i: (0, 0)),
        pl.BlockSpec((1, _D), lambda i: (0, 0)),
        pl.BlockSpec((1, 1, _NB), lambda i: (i, 0, 0)),
        pl.BlockSpec((_D, _D), lambda i: (0, 0)),
        pl.BlockSpec((1, _D), lambda i: (0, 0)),
        pl.BlockSpec((_NB, _D), lambda i: (i, 0)),
    ],
    out_specs=[
        pl.BlockSpec((_NB, _D), lambda i: (i, 0)),
        pl.BlockSpec((_G, _D), lambda i: (0, 0)),
        pl.BlockSpec((_NB, _D), lambda i: (i, 0)),
    ],
    out_shape=[
        jax.ShapeDtypeStruct((_N, _D), jnp.float32),
        jax.ShapeDtypeStruct((_G, _D), jnp.float32),
        jax.ShapeDtypeStruct((_N, _D), jnp.float32),
    ],
    scratch_shapes=[pltpu.VMEM((_G, _D), jnp.float32)],
)


def _bn_last_body(r_ref, st_ref, gamma_ref, beta_ref, batch_ref,
                  z_ref, g_ref, gacc):
    i = pl.program_id(0)
    z = _bn_math(r_ref[...], st_ref, gamma_ref, beta_ref)
    z_ref[...] = z

    @pl.when(i == 0)
    def _():
        gacc[...] = jnp.zeros_like(gacc)

    gacc[...] += _pool_block(z, batch_ref)

    @pl.when(i == _GRID - 1)
    def _():
        g_ref[...] = gacc[...]


_bn_last_call = pl.pallas_call(
    _bn_last_body,
    grid=(_GRID,),
    in_specs=[
        pl.BlockSpec((_NB, _D), lambda i: (i, 0)),
        pl.BlockSpec((8, _D), lambda i: (0, 0)),
        pl.BlockSpec((1, _D), lambda i: (0, 0)),
        pl.BlockSpec((1, _D), lambda i: (0, 0)),
        pl.BlockSpec((1, 1, _NB), lambda i: (i, 0, 0)),
    ],
    out_specs=[
        pl.BlockSpec((_NB, _D), lambda i: (i, 0)),
        pl.BlockSpec((_G, _D), lambda i: (0, 0)),
    ],
    out_shape=[
        jax.ShapeDtypeStruct((_N, _D), jnp.float32),
        jax.ShapeDtypeStruct((_G, _D), jnp.float32),
    ],
    scratch_shapes=[pltpu.VMEM((_G, _D), jnp.float32)],
)


def _edge_plan(edge_index):
    src2 = edge_index[0].reshape(_NC, _EH)
    dst2 = edge_index[1].reshape(_NC, _EH)
    h_idx = jnp.arange(_NW, dtype=jnp.int32) // _NS
    g_idx = jnp.arange(_NW, dtype=jnp.int32) % _NS
    src_rows = src2[h_idx] * 16 + g_idx[:, None]
    dst_rows = dst2[h_idx] * 8
    comb = jnp.stack(
        [src_rows.reshape(_NW, _NSC, _CH, _C),
         dst_rows.reshape(_NW, _NSC, _CH, _C)], axis=3)
    return comb


def _merge_partials(p):
    return (p.reshape(_NC, _NS, _NPAD, _FW)
            .transpose(0, 2, 1, 3)
            .reshape(_NC, _NPAD, _D)[:, :_N, :])


def kernel(x, edge_index, batch, W0, b0, bias0, gamma0, beta0,
           W1, b1, bias1, gamma1, beta1, W2, b2, bias2, gamma2, beta2):
    dstw = edge_index[1].reshape(_NW, _EW)
    comb = _edge_plan(edge_index)
    batch3 = batch.reshape(_GRID, 1, _NB)
    row = lambda v: v.reshape(1, _D)

    degp = _deg_call(dstw)
    deg = jnp.sum(degp, axis=0)[:_N] + 1.0
    disb = jnp.broadcast_to((deg ** -0.5)[:, None], (_N, _D))

    hp = _lin_call(x, W0.T, row(b0), disb)
    p = _merge_partials(_agg_call(hp.reshape(_N * _NS, _FW), comb))
    r, st = _conv_call(p, p, hp, disb, row(bias0))
    z0, g0, hp = _bn_call(r, st, row(gamma0), row(beta0), batch3,
                          W1.T, row(b1), disb)

    p = _merge_partials(_agg_call(hp.reshape(_N * _NS, _FW), comb))
    r, st = _conv_call(p, p, hp, disb, row(bias1))
    z1, g1, hp = _bn_call(r, st, row(gamma1), row(beta1), batch3,
                          W2.T, row(b2), disb)

    p = _merge_partials(_agg_call(hp.reshape(_N * _NS, _FW), comb))
    r, st = _conv_call(p, p, hp, disb, row(bias2))
    z2, g2 = _bn_last_call(r, st, row(gamma2), row(beta2), batch3)

    return (jnp.concatenate([z0, z1, z2], axis=1),
            jnp.concatenate([g0, g1, g2], axis=1))


# speedup vs baseline: 1.1506x; 1.1506x over previous
import functools

import jax
import jax.numpy as jnp
from jax import lax
from jax.experimental import pallas as pl
from jax.experimental.pallas import tpu as pltpu
from jax.experimental.pallas import tpu_sc as plsc

_N = 10000
_E = 320000
_D = 128
_G = 64
_NC = 2
_NS = 16
_NW = _NC * _NS
_FW = _D // _NS
_EH = _E // _NC
_C = 128
_CH = 50
_NSC = _EH // (_CH * _C)
_EW = _E // _NW
_NPAD = 10240
_TBL = _NPAD * _FW
_NB = 1000
_GRID = _N // _NB

_sc_mesh = plsc.VectorSubcoreMesh(core_axis_name="c", subcore_axis_name="s")
_sc_params = pltpu.CompilerParams(needs_layout_passes=False,
                                  use_tc_tiling_on_sc=False)


def _deg_body(dst_hbm, out_hbm, idx_v, table):
    c = lax.axis_index("c")
    s = lax.axis_index("s")
    wid = c * _NS + s

    def _zero(i, carry):
        table[pl.ds(i * 16, 16)] = jnp.zeros((16,), jnp.float32)
        return carry

    lax.fori_loop(0, _NPAD // 16, _zero, 0)
    pltpu.sync_copy(dst_hbm.at[wid], idx_v)
    ones = jnp.ones((16,), jnp.float32)

    def _count(k, carry):
        dst16 = idx_v[pl.ds(k * 16, 16)]
        plsc.addupdate_scatter(table, [dst16], ones)
        return carry

    lax.fori_loop(0, _EW // 16, _count, 0)
    pltpu.sync_copy(table, out_hbm.at[wid])


_deg_call = pl.kernel(
    _deg_body,
    out_type=jax.ShapeDtypeStruct((_NW, _NPAD), jnp.float32),
    mesh=_sc_mesh,
    compiler_params=_sc_params,
    scratch_types=[
        pltpu.VMEM((_EW,), jnp.int32),
        pltpu.VMEM((_NPAD,), jnp.float32),
    ],
)


def _agg_body(hp_hbm, comb_hbm, out_hbm, cbuf, rows0, rows1, tbl_a, tbl_b,
              sem0, sem1):
    c = lax.axis_index("c")
    s = lax.axis_index("s")
    wid = c * _NS + s

    def _zero(i, carry):
        tbl_a[pl.ds(i * 16, 16)] = jnp.zeros((16,), jnp.float32)
        tbl_b[pl.ds(i * 16, 16)] = jnp.zeros((16,), jnp.float32)
        return carry

    lax.fori_loop(0, _TBL // 32, _zero, 0)

    iota16 = lax.iota(jnp.int32, 16)

    def _compute(j, rows):
        for g in range(_C // 16):
            base4 = cbuf[j, 1, pl.ds(g * 16, 16)]
            row16 = g * 16 + iota16
            for f in range(_FW // 2):
                f16 = jnp.full((16,), f, jnp.int32)
                f16b = jnp.full((16,), f + 4, jnp.int32)
                vals_a = plsc.load_gather(rows, [row16, f16])
                vals_b = plsc.load_gather(rows, [row16, f16b])
                plsc.addupdate_scatter(tbl_a, [base4 + f], vals_a)
                plsc.addupdate_scatter(tbl_b, [base4 + f], vals_b)

    def _gather(j, rows, sem):
        return pltpu.async_copy(hp_hbm.at[cbuf.at[j, 0]], rows, sem)

    def _super(sc, carry):
        pltpu.sync_copy(comb_hbm.at[wid, sc], cbuf)
        _gather(0, rows0, sem0)

        def _pair(jj, carry2):
            a = 2 * jj
            _gather(a + 1, rows1, sem1)
            pltpu.make_async_copy(hp_hbm.at[cbuf.at[a, 0]], rows0, sem0).wait()
            _compute(a, rows0)

            @pl.when(jj < _CH // 2 - 1)
            def _():
                _gather(a + 2, rows0, sem0)

            pltpu.make_async_copy(
                hp_hbm.at[cbuf.at[a + 1, 0]], rows1, sem1).wait()
            _compute(a + 1, rows1)
            return carry2

        lax.fori_loop(0, _CH // 2, _pair, 0)
        return carry

    lax.fori_loop(0, _NSC, _super, 0)
    pltpu.sync_copy(tbl_a, out_hbm.at[c, s, 0])
    pltpu.sync_copy(tbl_b, out_hbm.at[c, s, 1])


_agg_call = pl.kernel(
    _agg_body,
    out_type=jax.ShapeDtypeStruct((_NC, _NS, 2, _TBL // 2), jnp.float32),
    mesh=_sc_mesh,
    compiler_params=_sc_params,
    scratch_types=[
        pltpu.VMEM((_CH, 2, _C), jnp.int32),
        pltpu.VMEM((_C, _FW), jnp.float32),
        pltpu.VMEM((_C, _FW), jnp.float32),
        pltpu.VMEM((_TBL // 2,), jnp.float32),
        pltpu.VMEM((_TBL // 2,), jnp.float32),
        pltpu.SemaphoreType.DMA,
        pltpu.SemaphoreType.DMA,
    ],
)


def _lin_body(x_ref, wt_ref, b_ref, disb_ref, hp_ref):
    h = jnp.dot(x_ref[...], wt_ref[...], preferred_element_type=jnp.float32)
    hp_ref[...] = (h + b_ref[...]) * disb_ref[...]


_lin_call = pl.pallas_call(
    _lin_body,
    grid=(_GRID,),
    in_specs=[
        pl.BlockSpec((_NB, _D), lambda i: (i, 0)),
        pl.BlockSpec((_D, _D), lambda i: (0, 0)),
        pl.BlockSpec((1, _D), lambda i: (0, 0)),
        pl.BlockSpec((_NB, _D), lambda i: (i, 0)),
    ],
    out_specs=pl.BlockSpec((_NB, _D), lambda i: (i, 0)),
    out_shape=jax.ShapeDtypeStruct((_N, _D), jnp.float32),
)


def _conv_body(p0_ref, p1_ref, hp_ref, disb_ref, bias_ref, r_ref, st_ref, acc):
    i = pl.program_id(0)
    conv = disb_ref[...] * (p0_ref[0] + p1_ref[0] + hp_ref[...]) + bias_ref[...]
    r = jnp.maximum(conv, 0.0)
    r_ref[...] = r

    @pl.when(i == 0)
    def _():
        acc[...] = jnp.zeros_like(acc)

    acc[0:1, :] += jnp.sum(r, axis=0, keepdims=True)
    acc[1:2, :] += jnp.sum(r * r, axis=0, keepdims=True)

    @pl.when(i == _GRID - 1)
    def _():
        st_ref[...] = acc[...]


_conv_call = pl.pallas_call(
    _conv_body,
    grid=(_GRID,),
    in_specs=[
        pl.BlockSpec((1, _NB, _D), lambda i: (0, i, 0)),
        pl.BlockSpec((1, _NB, _D), lambda i: (1, i, 0)),
        pl.BlockSpec((_NB, _D), lambda i: (i, 0)),
        pl.BlockSpec((_NB, _D), lambda i: (i, 0)),
        pl.BlockSpec((1, _D), lambda i: (0, 0)),
    ],
    out_specs=[
        pl.BlockSpec((_NB, _D), lambda i: (i, 0)),
        pl.BlockSpec((8, _D), lambda i: (0, 0)),
    ],
    out_shape=[
        jax.ShapeDtypeStruct((_N, _D), jnp.float32),
        jax.ShapeDtypeStruct((8, _D), jnp.float32),
    ],
    scratch_shapes=[pltpu.VMEM((8, _D), jnp.float32)],
)


def _bn_math(r, st_ref, gamma_ref, beta_ref):
    m = st_ref[0:1, :] * (1.0 / _N)
    var = st_ref[1:2, :] * (1.0 / _N) - m * m
    a = lax.rsqrt(var + 1e-5) * gamma_ref[...]
    return r * a + (beta_ref[...] - m * a)


def _pool_block(z, batch_ref):
    bt = batch_ref[0]
    ids = lax.broadcasted_iota(jnp.int32, (_G, 1), 0)
    onehot = (bt == ids).astype(jnp.float32)
    return jnp.dot(onehot, z, preferred_element_type=jnp.float32)


def _bn_body(r_ref, st_ref, gamma_ref, beta_ref, batch_ref, wt_ref, bn_ref,
             disb_ref, z_ref, g_ref, h_ref, gacc):
    i = pl.program_id(0)
    z = _bn_math(r_ref[...], st_ref, gamma_ref, beta_ref)
    z_ref[...] = z

    @pl.when(i == 0)
    def _():
        gacc[...] = jnp.zeros_like(gacc)

    gacc[...] += _pool_block(z, batch_ref)

    @pl.when(i == _GRID - 1)
    def _():
        g_ref[...] = gacc[...]

    h = jnp.dot(z, wt_ref[...], preferred_element_type=jnp.float32)
    h_ref[...] = (h + bn_ref[...]) * disb_ref[...]


_bn_call = pl.pallas_call(
    _bn_body,
    grid=(_GRID,),
    in_specs=[
        pl.BlockSpec((_NB, _D), lambda i: (i, 0)),
        pl.BlockSpec((8, _D), lambda i: (0, 0)),
        pl.BlockSpec((1, _D), lambda i: (0, 0)),
        pl.BlockSpec((1, _D), lambda i: (0, 0)),
        pl.BlockSpec((1, 1, _NB), lambda i: (i, 0, 0)),
        pl.BlockSpec((_D, _D), lambda i: (0, 0)),
        pl.BlockSpec((1, _D), lambda i: (0, 0)),
        pl.BlockSpec((_NB, _D), lambda i: (i, 0)),
    ],
    out_specs=[
        pl.BlockSpec((_NB, _D), lambda i: (i, 0)),
        pl.BlockSpec((_G, _D), lambda i: (0, 0)),
        pl.BlockSpec((_NB, _D), lambda i: (i, 0)),
    ],
    out_shape=[
        jax.ShapeDtypeStruct((_N, _D), jnp.float32),
        jax.ShapeDtypeStruct((_G, _D), jnp.float32),
        jax.ShapeDtypeStruct((_N, _D), jnp.float32),
    ],
    scratch_shapes=[pltpu.VMEM((_G, _D), jnp.float32)],
)


def _bn_last_body(r_ref, st_ref, gamma_ref, beta_ref, batch_ref,
                  z_ref, g_ref, gacc):
    i = pl.program_id(0)
    z = _bn_math(r_ref[...], st_ref, gamma_ref, beta_ref)
    z_ref[...] = z

    @pl.when(i == 0)
    def _():
        gacc[...] = jnp.zeros_like(gacc)

    gacc[...] += _pool_block(z, batch_ref)

    @pl.when(i == _GRID - 1)
    def _():
        g_ref[...] = gacc[...]


_bn_last_call = pl.pallas_call(
    _bn_last_body,
    grid=(_GRID,),
    in_specs=[
        pl.BlockSpec((_NB, _D), lambda i: (i, 0)),
        pl.BlockSpec((8, _D), lambda i: (0, 0)),
        pl.BlockSpec((1, _D), lambda i: (0, 0)),
        pl.BlockSpec((1, _D), lambda i: (0, 0)),
        pl.BlockSpec((1, 1, _NB), lambda i: (i, 0, 0)),
    ],
    out_specs=[
        pl.BlockSpec((_NB, _D), lambda i: (i, 0)),
        pl.BlockSpec((_G, _D), lambda i: (0, 0)),
    ],
    out_shape=[
        jax.ShapeDtypeStruct((_N, _D), jnp.float32),
        jax.ShapeDtypeStruct((_G, _D), jnp.float32),
    ],
    scratch_shapes=[pltpu.VMEM((_G, _D), jnp.float32)],
)


def _edge_plan(edge_index):
    src2 = edge_index[0].reshape(_NC, _EH)
    dst2 = edge_index[1].reshape(_NC, _EH)
    h_idx = jnp.arange(_NW, dtype=jnp.int32) // _NS
    g_idx = jnp.arange(_NW, dtype=jnp.int32) % _NS
    src_rows = src2[h_idx] * 16 + g_idx[:, None]
    dst_rows = dst2[h_idx] * 4
    comb = jnp.stack(
        [src_rows.reshape(_NW, _NSC, _CH, _C),
         dst_rows.reshape(_NW, _NSC, _CH, _C)], axis=3)
    return comb


def _merge_partials(p):
    return (p.reshape(_NC, _NS, 2, _NPAD, _FW // 2)
            .transpose(0, 3, 1, 2, 4)
            .reshape(_NC, _NPAD, _D)[:, :_N, :])


def kernel(x, edge_index, batch, W0, b0, bias0, gamma0, beta0,
           W1, b1, bias1, gamma1, beta1, W2, b2, bias2, gamma2, beta2):
    dstw = edge_index[1].reshape(_NW, _EW)
    comb = _edge_plan(edge_index)
    batch3 = batch.reshape(_GRID, 1, _NB)
    row = lambda v: v.reshape(1, _D)

    degp = _deg_call(dstw)
    deg = jnp.sum(degp, axis=0)[:_N] + 1.0
    disb = jnp.broadcast_to((deg ** -0.5)[:, None], (_N, _D))

    hp = _lin_call(x, W0.T, row(b0), disb)
    p = _merge_partials(_agg_call(hp.reshape(_N * _NS, _FW), comb))
    r, st = _conv_call(p, p, hp, disb, row(bias0))
    z0, g0, hp = _bn_call(r, st, row(gamma0), row(beta0), batch3,
                          W1.T, row(b1), disb)

    p = _merge_partials(_agg_call(hp.reshape(_N * _NS, _FW), comb))
    r, st = _conv_call(p, p, hp, disb, row(bias1))
    z1, g1, hp = _bn_call(r, st, row(gamma1), row(beta1), batch3,
                          W2.T, row(b2), disb)

    p = _merge_partials(_agg_call(hp.reshape(_N * _NS, _FW), comb))
    r, st = _conv_call(p, p, hp, disb, row(bias2))
    z2, g2 = _bn_last_call(r, st, row(gamma2), row(beta2), batch3)

    return (jnp.concatenate([z0, z1, z2], axis=1),
            jnp.concatenate([g0, g1, g2], axis=1))


# speedup vs baseline: 1.2551x; 1.0909x over previous
import functools

import jax
import jax.numpy as jnp
from jax import lax
from jax.experimental import pallas as pl
from jax.experimental.pallas import tpu as pltpu
from jax.experimental.pallas import tpu_sc as plsc

_N = 10000
_E = 320000
_D = 128
_G = 64
_NC = 2
_NS = 16
_NW = _NC * _NS
_FW = _D // _NS
_EH = _E // _NC
_C = 128
_CH = 50
_NSC = _EH // (_CH * _C)
_EW = _E // _NW
_NPAD = 10240
_TBL = _NPAD * _FW
_NB = 1000
_GRID = _N // _NB

_sc_mesh = plsc.VectorSubcoreMesh(core_axis_name="c", subcore_axis_name="s")
_sc_params = pltpu.CompilerParams(needs_layout_passes=False,
                                  use_tc_tiling_on_sc=False)


def _deg_body(dst_hbm, out_hbm, idx_v, table):
    c = lax.axis_index("c")
    s = lax.axis_index("s")
    wid = c * _NS + s

    def _zero(i, carry):
        table[pl.ds(i * 16, 16)] = jnp.zeros((16,), jnp.float32)
        return carry

    lax.fori_loop(0, _NPAD // 16, _zero, 0)
    pltpu.sync_copy(dst_hbm.at[wid], idx_v)
    ones = jnp.ones((16,), jnp.float32)

    def _count(k, carry):
        dst16 = idx_v[pl.ds(k * 16, 16)]
        plsc.addupdate_scatter(table, [dst16], ones)
        return carry

    lax.fori_loop(0, _EW // 16, _count, 0)
    pltpu.sync_copy(table, out_hbm.at[wid])


_deg_call = pl.kernel(
    _deg_body,
    out_type=jax.ShapeDtypeStruct((_NW, _NPAD), jnp.float32),
    mesh=_sc_mesh,
    compiler_params=_sc_params,
    scratch_types=[
        pltpu.VMEM((_EW,), jnp.int32),
        pltpu.VMEM((_NPAD,), jnp.float32),
    ],
)


def _agg_body(hp_hbm, comb_hbm, out_hbm, cbuf, rows0, rows1,
              tbl_a, tbl_b, tbl_c, tbl_d, sem0, sem1):
    c = lax.axis_index("c")
    s = lax.axis_index("s")
    wid = c * _NS + s
    tbls = (tbl_a, tbl_b, tbl_c, tbl_d)

    def _zero(i, carry):
        for t in tbls:
            t[pl.ds(i * 16, 16)] = jnp.zeros((16,), jnp.float32)
        return carry

    lax.fori_loop(0, _TBL // 64, _zero, 0)

    iota16 = lax.iota(jnp.int32, 16)

    def _compute(j, rows):
        for g in range(_C // 16):
            base2 = cbuf[j, 1, pl.ds(g * 16, 16)]
            row16 = g * 16 + iota16
            for f in range(2):
                vs = [plsc.load_gather(rows, [row16, jnp.full((16,), 2 * t + f,
                                                              jnp.int32)])
                      for t in range(4)]
                for t in range(4):
                    plsc.addupdate_scatter(tbls[t], [base2 + f], vs[t])

    def _gather(j, rows, sem):
        return pltpu.async_copy(hp_hbm.at[cbuf.at[j, 0]], rows, sem)

    def _super(sc, carry):
        pltpu.sync_copy(comb_hbm.at[wid, sc], cbuf)
        _gather(0, rows0, sem0)

        def _pair(jj, carry2):
            a = 2 * jj
            _gather(a + 1, rows1, sem1)
            pltpu.make_async_copy(hp_hbm.at[cbuf.at[a, 0]], rows0, sem0).wait()
            _compute(a, rows0)

            @pl.when(jj < _CH // 2 - 1)
            def _():
                _gather(a + 2, rows0, sem0)

            pltpu.make_async_copy(
                hp_hbm.at[cbuf.at[a + 1, 0]], rows1, sem1).wait()
            _compute(a + 1, rows1)
            return carry2

        lax.fori_loop(0, _CH // 2, _pair, 0)
        return carry

    lax.fori_loop(0, _NSC, _super, 0)
    for t in range(4):
        pltpu.sync_copy(tbls[t], out_hbm.at[c, s, t])


_agg_call = pl.kernel(
    _agg_body,
    out_type=jax.ShapeDtypeStruct((_NC, _NS, 4, _TBL // 4), jnp.float32),
    mesh=_sc_mesh,
    compiler_params=_sc_params,
    scratch_types=[
        pltpu.VMEM((_CH, 2, _C), jnp.int32),
        pltpu.VMEM((_C, _FW), jnp.float32),
        pltpu.VMEM((_C, _FW), jnp.float32),
        pltpu.VMEM((_TBL // 4,), jnp.float32),
        pltpu.VMEM((_TBL // 4,), jnp.float32),
        pltpu.VMEM((_TBL // 4,), jnp.float32),
        pltpu.VMEM((_TBL // 4,), jnp.float32),
        pltpu.SemaphoreType.DMA,
        pltpu.SemaphoreType.DMA,
    ],
)


def _lin_body(x_ref, wt_ref, b_ref, disb_ref, hp_ref):
    h = jnp.dot(x_ref[...], wt_ref[...], preferred_element_type=jnp.float32)
    hp_ref[...] = (h + b_ref[...]) * disb_ref[...]


_lin_call = pl.pallas_call(
    _lin_body,
    grid=(_GRID,),
    in_specs=[
        pl.BlockSpec((_NB, _D), lambda i: (i, 0)),
        pl.BlockSpec((_D, _D), lambda i: (0, 0)),
        pl.BlockSpec((1, _D), lambda i: (0, 0)),
        pl.BlockSpec((_NB, _D), lambda i: (i, 0)),
    ],
    out_specs=pl.BlockSpec((_NB, _D), lambda i: (i, 0)),
    out_shape=jax.ShapeDtypeStruct((_N, _D), jnp.float32),
)


def _conv_body(p0_ref, p1_ref, hp_ref, disb_ref, bias_ref, r_ref, st_ref, acc):
    i = pl.program_id(0)
    conv = disb_ref[...] * (p0_ref[0] + p1_ref[0] + hp_ref[...]) + bias_ref[...]
    r = jnp.maximum(conv, 0.0)
    r_ref[...] = r

    @pl.when(i == 0)
    def _():
        acc[...] = jnp.zeros_like(acc)

    acc[0:1, :] += jnp.sum(r, axis=0, keepdims=True)
    acc[1:2, :] += jnp.sum(r * r, axis=0, keepdims=True)

    @pl.when(i == _GRID - 1)
    def _():
        st_ref[...] = acc[...]


_conv_call = pl.pallas_call(
    _conv_body,
    grid=(_GRID,),
    in_specs=[
        pl.BlockSpec((1, _NB, _D), lambda i: (0, i, 0)),
        pl.BlockSpec((1, _NB, _D), lambda i: (1, i, 0)),
        pl.BlockSpec((_NB, _D), lambda i: (i, 0)),
        pl.BlockSpec((_NB, _D), lambda i: (i, 0)),
        pl.BlockSpec((1, _D), lambda i: (0, 0)),
    ],
    out_specs=[
        pl.BlockSpec((_NB, _D), lambda i: (i, 0)),
        pl.BlockSpec((8, _D), lambda i: (0, 0)),
    ],
    out_shape=[
        jax.ShapeDtypeStruct((_N, _D), jnp.float32),
        jax.ShapeDtypeStruct((8, _D), jnp.float32),
    ],
    scratch_shapes=[pltpu.VMEM((8, _D), jnp.float32)],
)


def _bn_math(r, st_ref, gamma_ref, beta_ref):
    m = st_ref[0:1, :] * (1.0 / _N)
    var = st_ref[1:2, :] * (1.0 / _N) - m * m
    a = lax.rsqrt(var + 1e-5) * gamma_ref[...]
    return r * a + (beta_ref[...] - m * a)


def _pool_block(z, batch_ref):
    bt = batch_ref[0]
    ids = lax.broadcasted_iota(jnp.int32, (_G, 1), 0)
    onehot = (bt == ids).astype(jnp.float32)
    return jnp.dot(onehot, z, preferred_element_type=jnp.float32)


def _bn_body(r_ref, st_ref, gamma_ref, beta_ref, batch_ref, wt_ref, bn_ref,
             disb_ref, z_ref, g_ref, h_ref, gacc):
    i = pl.program_id(0)
    z = _bn_math(r_ref[...], st_ref, gamma_ref, beta_ref)
    z_ref[...] = z

    @pl.when(i == 0)
    def _():
        gacc[...] = jnp.zeros_like(gacc)

    gacc[...] += _pool_block(z, batch_ref)

    @pl.when(i == _GRID - 1)
    def _():
        g_ref[...] = gacc[...]

    h = jnp.dot(z, wt_ref[...], preferred_element_type=jnp.float32)
    h_ref[...] = (h + bn_ref[...]) * disb_ref[...]


_bn_call = pl.pallas_call(
    _bn_body,
    grid=(_GRID,),
    in_specs=[
        pl.BlockSpec((_NB, _D), lambda i: (i, 0)),
        pl.BlockSpec((8, _D), lambda i: (0, 0)),
        pl.BlockSpec((1, _D), lambda i: (0, 0)),
        pl.BlockSpec((1, _D), lambda i: (0, 0)),
        pl.BlockSpec((1, 1, _NB), lambda i: (i, 0, 0)),
        pl.BlockSpec((_D, _D), lambda i: (0, 0)),
        pl.BlockSpec((1, _D), lambda i: (0, 0)),
        pl.BlockSpec((_NB, _D), lambda i: (i, 0)),
    ],
    out_specs=[
        pl.BlockSpec((_NB, _D), lambda i: (i, 0)),
        pl.BlockSpec((_G, _D), lambda i: (0, 0)),
        pl.BlockSpec((_NB, _D), lambda i: (i, 0)),
    ],
    out_shape=[
        jax.ShapeDtypeStruct((_N, _D), jnp.float32),
        jax.ShapeDtypeStruct((_G, _D), jnp.float32),
        jax.ShapeDtypeStruct((_N, _D), jnp.float32),
    ],
    scratch_shapes=[pltpu.VMEM((_G, _D), jnp.float32)],
)


def _bn_last_body(r_ref, st_ref, gamma_ref, beta_ref, batch_ref,
                  z_ref, g_ref, gacc):
    i = pl.program_id(0)
    z = _bn_math(r_ref[...], st_ref, gamma_ref, beta_ref)
    z_ref[...] = z

    @pl.when(i == 0)
    def _():
        gacc[...] = jnp.zeros_like(gacc)

    gacc[...] += _pool_block(z, batch_ref)

    @pl.when(i == _GRID - 1)
    def _():
        g_ref[...] = gacc[...]


_bn_last_call = pl.pallas_call(
    _bn_last_body,
    grid=(_GRID,),
    in_specs=[
        pl.BlockSpec((_NB, _D), lambda i: (i, 0)),
        pl.BlockSpec((8, _D), lambda i: (0, 0)),
        pl.BlockSpec((1, _D), lambda i: (0, 0)),
        pl.BlockSpec((1, _D), lambda i: (0, 0)),
        pl.BlockSpec((1, 1, _NB), lambda i: (i, 0, 0)),
    ],
    out_specs=[
        pl.BlockSpec((_NB, _D), lambda i: (i, 0)),
        pl.BlockSpec((_G, _D), lambda i: (0, 0)),
    ],
    out_shape=[
        jax.ShapeDtypeStruct((_N, _D), jnp.float32),
        jax.ShapeDtypeStruct((_G, _D), jnp.float32),
    ],
    scratch_shapes=[pltpu.VMEM((_G, _D), jnp.float32)],
)


def _edge_plan(edge_index):
    src2 = edge_index[0].reshape(_NC, _EH)
    dst2 = edge_index[1].reshape(_NC, _EH)
    h_idx = jnp.arange(_NW, dtype=jnp.int32) // _NS
    g_idx = jnp.arange(_NW, dtype=jnp.int32) % _NS
    src_rows = src2[h_idx] * 16 + g_idx[:, None]
    dst_rows = dst2[h_idx] * 2
    comb = jnp.stack(
        [src_rows.reshape(_NW, _NSC, _CH, _C),
         dst_rows.reshape(_NW, _NSC, _CH, _C)], axis=3)
    return comb


def _merge_partials(p):
    return (p.reshape(_NC, _NS, 4, _NPAD, _FW // 4)
            .transpose(0, 3, 1, 2, 4)
            .reshape(_NC, _NPAD, _D)[:, :_N, :])


def kernel(x, edge_index, batch, W0, b0, bias0, gamma0, beta0,
           W1, b1, bias1, gamma1, beta1, W2, b2, bias2, gamma2, beta2):
    dstw = edge_index[1].reshape(_NW, _EW)
    comb = _edge_plan(edge_index)
    batch3 = batch.reshape(_GRID, 1, _NB)
    row = lambda v: v.reshape(1, _D)

    degp = _deg_call(dstw)
    deg = jnp.sum(degp, axis=0)[:_N] + 1.0
    disb = jnp.broadcast_to((deg ** -0.5)[:, None], (_N, _D))

    hp = _lin_call(x, W0.T, row(b0), disb)
    p = _merge_partials(_agg_call(hp.reshape(_N * _NS, _FW), comb))
    r, st = _conv_call(p, p, hp, disb, row(bias0))
    z0, g0, hp = _bn_call(r, st, row(gamma0), row(beta0), batch3,
                          W1.T, row(b1), disb)

    p = _merge_partials(_agg_call(hp.reshape(_N * _NS, _FW), comb))
    r, st = _conv_call(p, p, hp, disb, row(bias1))
    z1, g1, hp = _bn_call(r, st, row(gamma1), row(beta1), batch3,
                          W2.T, row(b2), disb)

    p = _merge_partials(_agg_call(hp.reshape(_N * _NS, _FW), comb))
    r, st = _conv_call(p, p, hp, disb, row(bias2))
    z2, g2 = _bn_last_call(r, st, row(gamma2), row(beta2), batch3)

    return (jnp.concatenate([z0, z1, z2], axis=1),
            jnp.concatenate([g0, g1, g2], axis=1))


# speedup vs baseline: 1.3850x; 1.1034x over previous
import functools

import jax
import jax.numpy as jnp
from jax import lax
from jax.experimental import pallas as pl
from jax.experimental.pallas import tpu as pltpu
from jax.experimental.pallas import tpu_sc as plsc

_N = 10000
_E = 320000
_D = 128
_G = 64
_NC = 2
_NS = 16
_NW = _NC * _NS
_FW = _D // _NS
_EH = _E // _NC
_C = 128
_CH = 50
_NSC = _EH // (_CH * _C)
_EW = _E // _NW
_NPAD = 10240
_TBL = _NPAD * _FW
_NB = 1000
_GRID = _N // _NB

_sc_mesh = plsc.VectorSubcoreMesh(core_axis_name="c", subcore_axis_name="s")
_sc_params = pltpu.CompilerParams(needs_layout_passes=False,
                                  use_tc_tiling_on_sc=False)


def _deg_body(dst_hbm, out_hbm, idx_v, table):
    c = lax.axis_index("c")
    s = lax.axis_index("s")
    wid = c * _NS + s

    def _zero(i, carry):
        table[pl.ds(i * 16, 16)] = jnp.zeros((16,), jnp.float32)
        return carry

    lax.fori_loop(0, _NPAD // 16, _zero, 0)
    pltpu.sync_copy(dst_hbm.at[wid], idx_v)
    ones = jnp.ones((16,), jnp.float32)

    def _count(k, carry):
        dst16 = idx_v[pl.ds(k * 16, 16)]
        plsc.addupdate_scatter(table, [dst16], ones)
        return carry

    lax.fori_loop(0, _EW // 16, _count, 0)
    pltpu.sync_copy(table, out_hbm.at[wid])


_deg_call = pl.kernel(
    _deg_body,
    out_type=jax.ShapeDtypeStruct((_NW, _NPAD), jnp.float32),
    mesh=_sc_mesh,
    compiler_params=_sc_params,
    scratch_types=[
        pltpu.VMEM((_EW,), jnp.int32),
        pltpu.VMEM((_NPAD,), jnp.float32),
    ],
)


def _agg_body(hp_hbm, comb_hbm, out_hbm, cbuf, rows0, rows1,
              tbl_a, tbl_b, tbl_c, tbl_d, tbl_e, tbl_f, tbl_g, tbl_h,
              sem0, sem1):
    c = lax.axis_index("c")
    s = lax.axis_index("s")
    wid = c * _NS + s
    tbls = (tbl_a, tbl_b, tbl_c, tbl_d, tbl_e, tbl_f, tbl_g, tbl_h)

    def _zero(i, carry):
        for t in tbls:
            t[pl.ds(i * 16, 16)] = jnp.zeros((16,), jnp.float32)
        return carry

    lax.fori_loop(0, _NPAD // 16, _zero, 0)

    iota16 = lax.iota(jnp.int32, 16)

    def _compute(j, rows):
        for g in range(_C // 16):
            dst16 = cbuf[j, 1, pl.ds(g * 16, 16)]
            row16 = g * 16 + iota16
            vs = [plsc.load_gather(rows, [row16, jnp.full((16,), f, jnp.int32)])
                  for f in range(_FW)]
            for f in range(_FW):
                plsc.addupdate_scatter(tbls[f], [dst16], vs[f])

    def _gather(j, rows, sem):
        return pltpu.async_copy(hp_hbm.at[cbuf.at[j, 0]], rows, sem)

    def _super(sc, carry):
        pltpu.sync_copy(comb_hbm.at[wid, sc], cbuf)
        _gather(0, rows0, sem0)

        def _pair(jj, carry2):
            a = 2 * jj
            _gather(a + 1, rows1, sem1)
            pltpu.make_async_copy(hp_hbm.at[cbuf.at[a, 0]], rows0, sem0).wait()
            _compute(a, rows0)

            @pl.when(jj < _CH // 2 - 1)
            def _():
                _gather(a + 2, rows0, sem0)

            pltpu.make_async_copy(
                hp_hbm.at[cbuf.at[a + 1, 0]], rows1, sem1).wait()
            _compute(a + 1, rows1)
            return carry2

        lax.fori_loop(0, _CH // 2, _pair, 0)
        return carry

    lax.fori_loop(0, _NSC, _super, 0)
    for f in range(_FW):
        pltpu.sync_copy(tbls[f], out_hbm.at[c, s, f])


_agg_call = pl.kernel(
    _agg_body,
    out_type=jax.ShapeDtypeStruct((_NC, _NS, _FW, _NPAD), jnp.float32),
    mesh=_sc_mesh,
    compiler_params=_sc_params,
    scratch_types=[
        pltpu.VMEM((_CH, 2, _C), jnp.int32),
        pltpu.VMEM((_C, _FW), jnp.float32),
        pltpu.VMEM((_C, _FW), jnp.float32),
    ] + [pltpu.VMEM((_NPAD,), jnp.float32)] * _FW + [
        pltpu.SemaphoreType.DMA,
        pltpu.SemaphoreType.DMA,
    ],
)


def _lin_body(x_ref, wt_ref, b_ref, disb_ref, hp_ref):
    h = jnp.dot(x_ref[...], wt_ref[...], preferred_element_type=jnp.float32)
    hp_ref[...] = (h + b_ref[...]) * disb_ref[...]


_lin_call = pl.pallas_call(
    _lin_body,
    grid=(_GRID,),
    in_specs=[
        pl.BlockSpec((_NB, _D), lambda i: (i, 0)),
        pl.BlockSpec((_D, _D), lambda i: (0, 0)),
        pl.BlockSpec((1, _D), lambda i: (0, 0)),
        pl.BlockSpec((_NB, _D), lambda i: (i, 0)),
    ],
    out_specs=pl.BlockSpec((_NB, _D), lambda i: (i, 0)),
    out_shape=jax.ShapeDtypeStruct((_N, _D), jnp.float32),
)


def _conv_body(p0_ref, p1_ref, hp_ref, disb_ref, bias_ref, r_ref, st_ref, acc):
    i = pl.program_id(0)
    conv = disb_ref[...] * (p0_ref[0] + p1_ref[0] + hp_ref[...]) + bias_ref[...]
    r = jnp.maximum(conv, 0.0)
    r_ref[...] = r

    @pl.when(i == 0)
    def _():
        acc[...] = jnp.zeros_like(acc)

    acc[0:1, :] += jnp.sum(r, axis=0, keepdims=True)
    acc[1:2, :] += jnp.sum(r * r, axis=0, keepdims=True)

    @pl.when(i == _GRID - 1)
    def _():
        st_ref[...] = acc[...]


_conv_call = pl.pallas_call(
    _conv_body,
    grid=(_GRID,),
    in_specs=[
        pl.BlockSpec((1, _NB, _D), lambda i: (0, i, 0)),
        pl.BlockSpec((1, _NB, _D), lambda i: (1, i, 0)),
        pl.BlockSpec((_NB, _D), lambda i: (i, 0)),
        pl.BlockSpec((_NB, _D), lambda i: (i, 0)),
        pl.BlockSpec((1, _D), lambda i: (0, 0)),
    ],
    out_specs=[
        pl.BlockSpec((_NB, _D), lambda i: (i, 0)),
        pl.BlockSpec((8, _D), lambda i: (0, 0)),
    ],
    out_shape=[
        jax.ShapeDtypeStruct((_N, _D), jnp.float32),
        jax.ShapeDtypeStruct((8, _D), jnp.float32),
    ],
    scratch_shapes=[pltpu.VMEM((8, _D), jnp.float32)],
)


def _bn_math(r, st_ref, gamma_ref, beta_ref):
    m = st_ref[0:1, :] * (1.0 / _N)
    var = st_ref[1:2, :] * (1.0 / _N) - m * m
    a = lax.rsqrt(var + 1e-5) * gamma_ref[...]
    return r * a + (beta_ref[...] - m * a)


def _pool_block(z, batch_ref):
    bt = batch_ref[0]
    ids = lax.broadcasted_iota(jnp.int32, (_G, 1), 0)
    onehot = (bt == ids).astype(jnp.float32)
    return jnp.dot(onehot, z, preferred_element_type=jnp.float32)


def _bn_body(r_ref, st_ref, gamma_ref, beta_ref, batch_ref, wt_ref, bn_ref,
             disb_ref, z_ref, g_ref, h_ref, gacc):
    i = pl.program_id(0)
    z = _bn_math(r_ref[...], st_ref, gamma_ref, beta_ref)
    z_ref[...] = z

    @pl.when(i == 0)
    def _():
        gacc[...] = jnp.zeros_like(gacc)

    gacc[...] += _pool_block(z, batch_ref)

    @pl.when(i == _GRID - 1)
    def _():
        g_ref[...] = gacc[...]

    h = jnp.dot(z, wt_ref[...], preferred_element_type=jnp.float32)
    h_ref[...] = (h + bn_ref[...]) * disb_ref[...]


_bn_call = pl.pallas_call(
    _bn_body,
    grid=(_GRID,),
    in_specs=[
        pl.BlockSpec((_NB, _D), lambda i: (i, 0)),
        pl.BlockSpec((8, _D), lambda i: (0, 0)),
        pl.BlockSpec((1, _D), lambda i: (0, 0)),
        pl.BlockSpec((1, _D), lambda i: (0, 0)),
        pl.BlockSpec((1, 1, _NB), lambda i: (i, 0, 0)),
        pl.BlockSpec((_D, _D), lambda i: (0, 0)),
        pl.BlockSpec((1, _D), lambda i: (0, 0)),
        pl.BlockSpec((_NB, _D), lambda i: (i, 0)),
    ],
    out_specs=[
        pl.BlockSpec((_NB, _D), lambda i: (i, 0)),
        pl.BlockSpec((_G, _D), lambda i: (0, 0)),
        pl.BlockSpec((_NB, _D), lambda i: (i, 0)),
    ],
    out_shape=[
        jax.ShapeDtypeStruct((_N, _D), jnp.float32),
        jax.ShapeDtypeStruct((_G, _D), jnp.float32),
        jax.ShapeDtypeStruct((_N, _D), jnp.float32),
    ],
    scratch_shapes=[pltpu.VMEM((_G, _D), jnp.float32)],
)


def _bn_last_body(r_ref, st_ref, gamma_ref, beta_ref, batch_ref,
                  z_ref, g_ref, gacc):
    i = pl.program_id(0)
    z = _bn_math(r_ref[...], st_ref, gamma_ref, beta_ref)
    z_ref[...] = z

    @pl.when(i == 0)
    def _():
        gacc[...] = jnp.zeros_like(gacc)

    gacc[...] += _pool_block(z, batch_ref)

    @pl.when(i == _GRID - 1)
    def _():
        g_ref[...] = gacc[...]


_bn_last_call = pl.pallas_call(
    _bn_last_body,
    grid=(_GRID,),
    in_specs=[
        pl.BlockSpec((_NB, _D), lambda i: (i, 0)),
        pl.BlockSpec((8, _D), lambda i: (0, 0)),
        pl.BlockSpec((1, _D), lambda i: (0, 0)),
        pl.BlockSpec((1, _D), lambda i: (0, 0)),
        pl.BlockSpec((1, 1, _NB), lambda i: (i, 0, 0)),
    ],
    out_specs=[
        pl.BlockSpec((_NB, _D), lambda i: (i, 0)),
        pl.BlockSpec((_G, _D), lambda i: (0, 0)),
    ],
    out_shape=[
        jax.ShapeDtypeStruct((_N, _D), jnp.float32),
        jax.ShapeDtypeStruct((_G, _D), jnp.float32),
    ],
    scratch_shapes=[pltpu.VMEM((_G, _D), jnp.float32)],
)


def _edge_plan(edge_index):
    src2 = edge_index[0].reshape(_NC, _EH)
    dst2 = edge_index[1].reshape(_NC, _EH)
    h_idx = jnp.arange(_NW, dtype=jnp.int32) // _NS
    g_idx = jnp.arange(_NW, dtype=jnp.int32) % _NS
    src_rows = src2[h_idx] * 16 + g_idx[:, None]
    dst_rows = dst2[h_idx] + jnp.zeros((_NW, 1), jnp.int32)
    comb = jnp.stack(
        [src_rows.reshape(_NW, _NSC, _CH, _C),
         dst_rows.reshape(_NW, _NSC, _CH, _C)], axis=3)
    return comb


def _merge_partials(p):
    return (p.reshape(_NC, _NS, _FW, _NPAD)
            .transpose(0, 3, 1, 2)
            .reshape(_NC, _NPAD, _D)[:, :_N, :])


def kernel(x, edge_index, batch, W0, b0, bias0, gamma0, beta0,
           W1, b1, bias1, gamma1, beta1, W2, b2, bias2, gamma2, beta2):
    dstw = edge_index[1].reshape(_NW, _EW)
    comb = _edge_plan(edge_index)
    batch3 = batch.reshape(_GRID, 1, _NB)
    row = lambda v: v.reshape(1, _D)

    degp = _deg_call(dstw)
    deg = jnp.sum(degp, axis=0)[:_N] + 1.0
    disb = jnp.broadcast_to((deg ** -0.5)[:, None], (_N, _D))

    hp = _lin_call(x, W0.T, row(b0), disb)
    p = _merge_partials(_agg_call(hp.reshape(_N * _NS, _FW), comb))
    r, st = _conv_call(p, p, hp, disb, row(bias0))
    z0, g0, hp = _bn_call(r, st, row(gamma0), row(beta0), batch3,
                          W1.T, row(b1), disb)

    p = _merge_partials(_agg_call(hp.reshape(_N * _NS, _FW), comb))
    r, st = _conv_call(p, p, hp, disb, row(bias1))
    z1, g1, hp = _bn_call(r, st, row(gamma1), row(beta1), batch3,
                          W2.T, row(b2), disb)

    p = _merge_partials(_agg_call(hp.reshape(_N * _NS, _FW), comb))
    r, st = _conv_call(p, p, hp, disb, row(bias2))
    z2, g2 = _bn_last_call(r, st, row(gamma2), row(beta2), batch3)

    return (jnp.concatenate([z0, z1, z2], axis=1),
            jnp.concatenate([g0, g1, g2], axis=1))
